# baseline (speedup 1.0000x reference)
import functools

import jax
import jax.numpy as jnp
from jax import lax
from jax.experimental import pallas as pl
from jax.experimental.pallas import tpu as pltpu
from jax.experimental.pallas import tpu_sc as plsc

_B = 16384
_D = 64
_NC = 2
_NS = 16
_NW = _NC * _NS
_TOT = 2 * _B
_PER_W = _TOT // _NW
_CHUNK = 128
_NCHUNK = _PER_W // _CHUNK
_EPS = 1e-5


def _sc_gather(table, idx2d):
    mesh = plsc.VectorSubcoreMesh(core_axis_name="c", subcore_axis_name="s")

    @functools.partial(
        pl.kernel,
        mesh=mesh,
        out_type=jax.ShapeDtypeStruct((_TOT, _D), jnp.float32),
        scratch_types=[
            pltpu.VMEM((_NCHUNK, _CHUNK), jnp.int32),
            pltpu.VMEM((_PER_W, _D), jnp.float32),
            pltpu.SemaphoreType.DMA,
        ],
        compiler_params=pltpu.CompilerParams(use_tc_tiling_on_sc=False),
    )
    def k(table_hbm, idx_hbm, out_hbm, idx_v, rows_v, sem):
        wid = lax.axis_index("s") * _NC + lax.axis_index("c")
        pltpu.sync_copy(idx_hbm.at[pl.ds(wid * _NCHUNK, _NCHUNK)], idx_v)
        copies = []
        for j in range(_NCHUNK):
            copies.append(
                pltpu.async_copy(
                    table_hbm.at[idx_v.at[j]],
                    rows_v.at[pl.ds(j * _CHUNK, _CHUNK)],
                    sem,
                )
            )
        for c in copies:
            c.wait()
        pltpu.sync_copy(rows_v, out_hbm.at[pl.ds(wid * _PER_W, _PER_W)])

    return k(table, idx2d)


def _tc_body(e_ref, g1_ref, b1_ref, g2_ref, b2_ref, out_ref):
    e1 = e_ref[0:_B, :]
    e2 = e_ref[_B:, :]
    m1 = jnp.mean(e1, axis=0, keepdims=True)
    m2 = jnp.mean(e2, axis=0, keepdims=True)
    d1 = e1 - m1
    d2 = e2 - m2
    v1 = jnp.mean(d1 * d1, axis=0, keepdims=True)
    v2 = jnp.mean(d2 * d2, axis=0, keepdims=True)
    a1 = g1_ref[...] * lax.rsqrt(v1 + _EPS)
    a2 = g2_ref[...] * lax.rsqrt(v2 + _EPS)
    n1 = d1 * a1 + b1_ref[...]
    n2 = d2 * a2 + b2_ref[...]
    ones = jnp.ones((_D,), dtype=jnp.float32)
    s = jax.lax.dot_general(
        ones, n1 * n2, (((0,), (1,)), ((), ())),
        preferred_element_type=jnp.float32)
    out_ref[...] = jax.nn.sigmoid(s)


def _tc_compute(e, g1, b1, g2, b2):
    return pl.pallas_call(
        _tc_body,
        out_shape=jax.ShapeDtypeStruct((_B,), jnp.float32),
    )(e, g1, b1, g2, b2)


def kernel(idx, table, gamma1, beta1, gamma2, beta2):
    idx2d = idx.T.reshape(_NW * _NCHUNK, _CHUNK).astype(jnp.int32)
    e = _sc_gather(table, idx2d)
    return _tc_compute(
        e,
        gamma1.reshape(1, _D), beta1.reshape(1, _D),
        gamma2.reshape(1, _D), beta2.reshape(1, _D),
    )
